# baseline (speedup 1.0000x reference)
import jax
import jax.numpy as jnp
from jax import lax
from jax.experimental import pallas as pl
from jax.experimental.pallas import tpu as pltpu
from jax.experimental.pallas import tpu_sc as plsc

_NC = 2
_EI_R = 2 * 6400000 // 128
_EW_R = 6400000 // 128
_EI_PC = _EI_R // _NC
_EW_PC = _EW_R // _NC
_STEPS = 25
_NBUF = 4
_DELAY = 2
_EI_CH = _EI_PC // _STEPS
_EW_CH = _EW_PC // _STEPS


def _sc_copy_body(ei_in, ew_in, ei_out, ew_out,
                  ei_buf0, ei_buf1, ei_buf2, ei_buf3,
                  ew_buf0, ew_buf1, ew_buf2, ew_buf3,
                  sei_in, sei_out, sew_in, sew_out):
    cid = lax.axis_index("c")
    ei_base = cid * _EI_PC
    ew_base = cid * _EW_PC
    ei_bufs = (ei_buf0, ei_buf1, ei_buf2, ei_buf3)
    ew_bufs = (ew_buf0, ew_buf1, ew_buf2, ew_buf3)

    def copies(t):
        b = t % _NBUF
        ei_sl = pl.ds(ei_base + t * _EI_CH, _EI_CH)
        ew_sl = pl.ds(ew_base + t * _EW_CH, _EW_CH)
        return (
            pltpu.make_async_copy(ei_in.at[ei_sl], ei_bufs[b], sei_in.at[b]),
            pltpu.make_async_copy(ew_in.at[ew_sl], ew_bufs[b], sew_in.at[b]),
            pltpu.make_async_copy(ei_bufs[b], ei_out.at[ei_sl], sei_out.at[b]),
            pltpu.make_async_copy(ew_bufs[b], ew_out.at[ew_sl], sew_out.at[b]),
        )

    for c in range(min(_NBUF, _STEPS)):
        cei_in, cew_in, _, _ = copies(c)
        cei_in.start()
        cew_in.start()
    for t in range(_STEPS):
        j = t - _DELAY
        if j >= 0 and j + _NBUF < _STEPS:
            _, _, pei_out, pew_out = copies(j)
            pei_out.wait()
            pew_out.wait()
            nei_in, new_in, _, _ = copies(j + _NBUF)
            nei_in.start()
            new_in.start()
        cei_in, cew_in, cei_out, cew_out = copies(t)
        cei_in.wait()
        cew_in.wait()
        cei_out.start()
        cew_out.start()
    for t in range(max(0, _STEPS - _NBUF), _STEPS):
        _, _, cei_out, cew_out = copies(t)
        cei_out.wait()
        cew_out.wait()


_sc_copy = pl.kernel(
    _sc_copy_body,
    out_type=(
        jax.ShapeDtypeStruct((_EI_R, 128), jnp.int32),
        jax.ShapeDtypeStruct((_EW_R, 128), jnp.float32),
    ),
    mesh=plsc.ScalarSubcoreMesh(axis_name="c", num_cores=_NC),
    scratch_types=(
        pltpu.VMEM_SHARED((_EI_CH, 128), jnp.int32),
        pltpu.VMEM_SHARED((_EI_CH, 128), jnp.int32),
        pltpu.VMEM_SHARED((_EI_CH, 128), jnp.int32),
        pltpu.VMEM_SHARED((_EI_CH, 128), jnp.int32),
        pltpu.VMEM_SHARED((_EW_CH, 128), jnp.float32),
        pltpu.VMEM_SHARED((_EW_CH, 128), jnp.float32),
        pltpu.VMEM_SHARED((_EW_CH, 128), jnp.float32),
        pltpu.VMEM_SHARED((_EW_CH, 128), jnp.float32),
        pltpu.SemaphoreType.DMA((_NBUF,)),
        pltpu.SemaphoreType.DMA((_NBUF,)),
        pltpu.SemaphoreType.DMA((_NBUF,)),
        pltpu.SemaphoreType.DMA((_NBUF,)),
    ),
)


def kernel(ei, ew):
    ei_flat, ew_flat = _sc_copy(ei.reshape(_EI_R, 128), ew.reshape(_EW_R, 128))
    return ei_flat.reshape(ei.shape), ew_flat.reshape(ew.shape)


# speedup vs baseline: 1.0509x; 1.0509x over previous
import jax
import jax.numpy as jnp
from jax import lax
from jax.experimental import pallas as pl
from jax.experimental.pallas import tpu as pltpu
from jax.experimental.pallas import tpu_sc as plsc


_K = 25
_TNBUF = 6
_TDELAY = 2


def _tc_copy_body(ei_ref, ei_out, ei_buf, s_in, s_out):
    re_ = ei_ref.shape[0] // _K

    def in_copy(k):
        s = k % _TNBUF
        return pltpu.make_async_copy(
            ei_ref.at[pl.ds(k * re_, re_), :], ei_buf.at[s], s_in.at[s])

    def out_copy(k):
        s = k % _TNBUF
        return pltpu.make_async_copy(
            ei_buf.at[s], ei_out.at[pl.ds(k * re_, re_), :], s_out.at[s])

    for k in range(min(_TNBUF, _K)):
        in_copy(k).start()
    for k in range(_K):
        in_copy(k).wait()
        out_copy(k).start()
        j = k - _TDELAY
        if j >= 0 and j + _TNBUF < _K:
            out_copy(j).wait()
            in_copy(j + _TNBUF).start()
    for j in range(max(0, _K - _TNBUF), _K):
        out_copy(j).wait()


def _tc_copy(ei2):
    re_ = ei2.shape[0] // _K
    return pl.pallas_call(
        _tc_copy_body,
        in_specs=(pl.BlockSpec(memory_space=pl.ANY),),
        out_specs=pl.BlockSpec(memory_space=pl.ANY),
        out_shape=jax.ShapeDtypeStruct(ei2.shape, ei2.dtype),
        scratch_shapes=(
            pltpu.VMEM((_TNBUF, re_, 128), ei2.dtype),
            pltpu.SemaphoreType.DMA((_TNBUF,)),
            pltpu.SemaphoreType.DMA((_TNBUF,)),
        ),
        compiler_params=pltpu.CompilerParams(
            vmem_limit_bytes=60 * 1024 * 1024,
        ),
    )(ei2)


_NC = 2
_EW_R = 6400000 // 128
_EW_PC = _EW_R // _NC
_STEPS = 25
_SNBUF = 4
_SDELAY = 2
_EW_CH = _EW_PC // _STEPS


def _sc_copy_body(ew_in, ew_out, b0, b1, b2, b3, s_in, s_out):
    cid = lax.axis_index("c")
    base = cid * _EW_PC
    bufs = (b0, b1, b2, b3)

    def copies(t):
        b = t % _SNBUF
        sl = pl.ds(base + t * _EW_CH, _EW_CH)
        return (
            pltpu.make_async_copy(ew_in.at[sl], bufs[b], s_in.at[b]),
            pltpu.make_async_copy(bufs[b], ew_out.at[sl], s_out.at[b]),
        )

    for c in range(min(_SNBUF, _STEPS)):
        cin, _ = copies(c)
        cin.start()
    for t in range(_STEPS):
        j = t - _SDELAY
        if j >= 0 and j + _SNBUF < _STEPS:
            _, pout = copies(j)
            pout.wait()
            nin, _ = copies(j + _SNBUF)
            nin.start()
        cin, cout = copies(t)
        cin.wait()
        cout.start()
    for t in range(max(0, _STEPS - _SNBUF), _STEPS):
        _, cout = copies(t)
        cout.wait()


_sc_copy = pl.kernel(
    _sc_copy_body,
    out_type=jax.ShapeDtypeStruct((_EW_R, 128), jnp.float32),
    mesh=plsc.ScalarSubcoreMesh(axis_name="c", num_cores=_NC),
    scratch_types=(
        pltpu.VMEM_SHARED((_EW_CH, 128), jnp.float32),
        pltpu.VMEM_SHARED((_EW_CH, 128), jnp.float32),
        pltpu.VMEM_SHARED((_EW_CH, 128), jnp.float32),
        pltpu.VMEM_SHARED((_EW_CH, 128), jnp.float32),
        pltpu.SemaphoreType.DMA((_SNBUF,)),
        pltpu.SemaphoreType.DMA((_SNBUF,)),
    ),
)


def kernel(ei, ew):
    ei2 = ei.reshape(ei.size // 128, 128)
    ew2 = ew.reshape(_EW_R, 128)
    ei_c = _tc_copy(ei2)
    ew_c = _sc_copy(ew2)
    return ei_c.reshape(ei.shape), ew_c.reshape(ew.shape)


# speedup vs baseline: 1.0821x; 1.0297x over previous
import jax
import jax.numpy as jnp
from jax.experimental import pallas as pl
from jax.experimental.pallas import tpu as pltpu

_K = 25
_NBUF = 6
_DELAY = 2


def _copy_body(ei_ref, ew_ref, ei_out, ew_out,
               ei_buf, ew_buf, sei_in, sei_out, sew_in, sew_out):
    re_ = ei_ref.shape[0] // _K
    rw = ew_ref.shape[0] // _K

    def in_copies(k):
        s = k % _NBUF
        return (
            pltpu.make_async_copy(
                ei_ref.at[pl.ds(k * re_, re_), :], ei_buf.at[s], sei_in.at[s]),
            pltpu.make_async_copy(
                ew_ref.at[pl.ds(k * rw, rw), :], ew_buf.at[s], sew_in.at[s]),
        )

    def out_copies(k):
        s = k % _NBUF
        return (
            pltpu.make_async_copy(
                ei_buf.at[s], ei_out.at[pl.ds(k * re_, re_), :], sei_out.at[s]),
            pltpu.make_async_copy(
                ew_buf.at[s], ew_out.at[pl.ds(k * rw, rw), :], sew_out.at[s]),
        )

    for k in range(min(_NBUF, _K)):
        for c in in_copies(k):
            c.start()
    for k in range(_K):
        for c in in_copies(k):
            c.wait()
        for c in out_copies(k):
            c.start(priority=1)
        j = k - _DELAY
        if j >= 0 and j + _NBUF < _K:
            for c in out_copies(j):
                c.wait()
            for c in in_copies(j + _NBUF):
                c.start()
    for j in range(max(0, _K - _NBUF), _K):
        for c in out_copies(j):
            c.wait()


def kernel(ei, ew):
    ei2 = ei.reshape(ei.size // 128, 128)
    ew2 = ew.reshape(ew.size // 128, 128)
    re_ = ei2.shape[0] // _K
    rw = ew2.shape[0] // _K
    out = pl.pallas_call(
        _copy_body,
        in_specs=(
            pl.BlockSpec(memory_space=pl.ANY),
            pl.BlockSpec(memory_space=pl.ANY),
        ),
        out_specs=(
            pl.BlockSpec(memory_space=pl.ANY),
            pl.BlockSpec(memory_space=pl.ANY),
        ),
        out_shape=(
            jax.ShapeDtypeStruct(ei2.shape, ei2.dtype),
            jax.ShapeDtypeStruct(ew2.shape, ew2.dtype),
        ),
        scratch_shapes=(
            pltpu.VMEM((_NBUF, re_, 128), ei.dtype),
            pltpu.VMEM((_NBUF, rw, 128), ew.dtype),
            pltpu.SemaphoreType.DMA((_NBUF,)),
            pltpu.SemaphoreType.DMA((_NBUF,)),
            pltpu.SemaphoreType.DMA((_NBUF,)),
            pltpu.SemaphoreType.DMA((_NBUF,)),
        ),
    )(ei2, ew2)
    return out[0].reshape(ei.shape), out[1].reshape(ew.shape)
